# baseline (speedup 1.0000x reference)
import functools

import jax
import jax.numpy as jnp
from jax import lax
from jax.experimental import pallas as pl
from jax.experimental.pallas import tpu as pltpu
from jax.experimental.pallas import tpu_sc as plsc

_EMBED_DIM = 512

_mesh = plsc.ScalarSubcoreMesh(axis_name="c", num_cores=2)


@functools.partial(
    pl.kernel,
    out_type=(
        jax.ShapeDtypeStruct((_EMBED_DIM,), jnp.float32),
        jax.ShapeDtypeStruct((_EMBED_DIM,), jnp.float32),
        jax.ShapeDtypeStruct((_EMBED_DIM,), jnp.float32),
    ),
    mesh=_mesh,
    scratch_types=[
        pltpu.SMEM((1,), jnp.int32),
        pltpu.SMEM((1,), jnp.int32),
        pltpu.SemaphoreType.DMA,
        pltpu.SemaphoreType.DMA,
    ],
)
def _lookup(src_hbm, edge_hbm, dst_hbm, node_tab, edge_tab,
            src_out, edge_out, dst_out,
            id0_s, id1_s, sem0, sem1):
    cid = lax.axis_index("c")

    @pl.when(cid == 0)
    def _node_core():
        i0 = pltpu.async_copy(src_hbm, id0_s, sem0)
        i1 = pltpu.async_copy(dst_hbm, id1_s, sem1)
        i0.wait()
        c0 = pltpu.async_copy(node_tab.at[id0_s[0]], src_out, sem0)
        i1.wait()
        c1 = pltpu.async_copy(node_tab.at[id1_s[0]], dst_out, sem1)
        c0.wait()
        c1.wait()

    @pl.when(cid == 1)
    def _edge_core():
        pltpu.async_copy(edge_hbm, id0_s, sem0).wait()
        pltpu.async_copy(edge_tab.at[id0_s[0]], edge_out, sem0).wait()


def kernel(src_type, edge_type, dst_type, node_type_embed, edge_type_embed):
    src = jnp.asarray(src_type, jnp.int32).reshape((1,))
    edge = jnp.asarray(edge_type, jnp.int32).reshape((1,))
    dst = jnp.asarray(dst_type, jnp.int32).reshape((1,))
    src_embed, edge_embed, dst_embed = _lookup(
        src, edge, dst, node_type_embed, edge_type_embed)
    return (src_embed, edge_embed, dst_embed)


# speedup vs baseline: 1.0771x; 1.0771x over previous
import functools

import jax
import jax.numpy as jnp
from jax.experimental import pallas as pl
from jax.experimental.pallas import tpu as pltpu
from jax.experimental.pallas import tpu_sc as plsc

_EMBED_DIM = 512

_mesh = plsc.ScalarSubcoreMesh(axis_name="c", num_cores=1)


@functools.partial(
    pl.kernel,
    out_type=(
        jax.ShapeDtypeStruct((_EMBED_DIM,), jnp.float32),
        jax.ShapeDtypeStruct((_EMBED_DIM,), jnp.float32),
        jax.ShapeDtypeStruct((_EMBED_DIM,), jnp.float32),
    ),
    mesh=_mesh,
    scratch_types=[
        pltpu.SMEM((1,), jnp.int32),
        pltpu.SMEM((1,), jnp.int32),
        pltpu.SMEM((1,), jnp.int32),
        pltpu.SemaphoreType.DMA,
        pltpu.SemaphoreType.DMA,
        pltpu.SemaphoreType.DMA,
    ],
)
def _lookup(src_hbm, edge_hbm, dst_hbm, node_tab, edge_tab,
            src_out, edge_out, dst_out,
            src_s, edge_s, dst_s, sem0, sem1, sem2):
    i0 = pltpu.async_copy(src_hbm, src_s, sem0)
    i1 = pltpu.async_copy(edge_hbm, edge_s, sem1)
    i2 = pltpu.async_copy(dst_hbm, dst_s, sem2)
    i0.wait()
    c0 = pltpu.async_copy(node_tab.at[src_s[0]], src_out, sem0)
    i1.wait()
    c1 = pltpu.async_copy(edge_tab.at[edge_s[0]], edge_out, sem1)
    i2.wait()
    c2 = pltpu.async_copy(node_tab.at[dst_s[0]], dst_out, sem2)
    c0.wait()
    c1.wait()
    c2.wait()


def kernel(src_type, edge_type, dst_type, node_type_embed, edge_type_embed):
    src = jnp.asarray(src_type, jnp.int32).reshape((1,))
    edge = jnp.asarray(edge_type, jnp.int32).reshape((1,))
    dst = jnp.asarray(dst_type, jnp.int32).reshape((1,))
    src_embed, edge_embed, dst_embed = _lookup(
        src, edge, dst, node_type_embed, edge_type_embed)
    return (src_embed, edge_embed, dst_embed)
